# Initial kernel scaffold; baseline (speedup 1.0000x reference)
import functools

import jax
import jax.numpy as jnp
from jax import lax
from jax.experimental import pallas as pl
from jax.experimental.pallas import tpu as pltpu

E = 8
TOP_K = 2
DIM = 1024
INTER = 512
NV = 10


def _silu(v):
    return v * (1.0 / (1.0 + jnp.exp(-v)))


def _moe_block(x_ref, gw_ref, gb_ref, v1_ref, vb1_ref, v2_ref, vb2_ref,
               v3_ref, vb3_ref, out_ref, comb_ref):
    e = pl.program_id(1)
    xs = x_ref[...]

    @pl.when(e == 0)
    def _gate():
        logits = lax.dot_general(xs, gw_ref[...], (((1,), (1,)), ((), ())),
                                 preferred_element_type=jnp.float32)
        logits = logits + gb_ref[...]
        m = jnp.max(logits, axis=1, keepdims=True)
        ex = jnp.exp(logits - m)
        scores = ex / jnp.sum(ex, axis=1, keepdims=True)
        cols = lax.broadcasted_iota(jnp.int32, scores.shape, 1)
        m1 = jnp.max(scores, axis=1, keepdims=True)
        i1 = jnp.min(jnp.where(scores == m1, cols, E), axis=1, keepdims=True)
        masked = jnp.where(cols == i1, -jnp.inf, scores)
        m2 = jnp.max(masked, axis=1, keepdims=True)
        i2 = jnp.min(jnp.where(masked == m2, cols, E), axis=1, keepdims=True)
        denom = m1 + m2 + 1e-20
        w1 = m1 / denom
        w2 = m2 / denom
        cols16 = lax.broadcasted_iota(jnp.int32, (xs.shape[0], 16), 1)
        comb = (jnp.where(cols16 == i1, w1, 0.0)
                + jnp.where(cols16 == i2, w2, 0.0)
                + jnp.where(cols16 >= E, 1.0, 0.0))
        comb_ref[...] = comb

    h1 = lax.dot_general(xs, v1_ref[0], (((1,), (1,)), ((), ())),
                         preferred_element_type=jnp.float32) + vb1_ref[...]
    h3 = lax.dot_general(xs, v3_ref[0], (((1,), (1,)), ((), ())),
                         preferred_element_type=jnp.float32) + vb3_ref[...]
    h = _silu(h1) * h3
    oe = lax.dot_general(h, v2_ref[0], (((1,), (1,)), ((), ())),
                         preferred_element_type=jnp.float32) + vb2_ref[...]
    cols16 = lax.broadcasted_iota(jnp.int32, comb_ref.shape, 1)
    ce = jnp.sum(jnp.where(cols16 == e, comb_ref[...], 0.0), axis=1,
                 keepdims=True)

    @pl.when(e == 0)
    def _init():
        out_ref[...] = oe * ce

    @pl.when(e != 0)
    def _acc():
        out_ref[...] = out_ref[...] + oe * ce


@functools.partial(jax.jit, static_argnames=("bm",))
def _moe_dense(xf, gate_w, gate_b, V1, VB1, V2, VB2, V3, VB3, bm=1024):
    t = xf.shape[0]
    grid = (t // bm, NV)
    return pl.pallas_call(
        _moe_block,
        grid=grid,
        in_specs=[
            pl.BlockSpec((bm, DIM), lambda i, e: (i, 0)),
            pl.BlockSpec((E, DIM), lambda i, e: (0, 0)),
            pl.BlockSpec((E,), lambda i, e: (0,)),
            pl.BlockSpec((1, INTER, DIM), lambda i, e: (e, 0, 0)),
            pl.BlockSpec((1, INTER), lambda i, e: (e, 0)),
            pl.BlockSpec((1, DIM, INTER), lambda i, e: (e, 0, 0)),
            pl.BlockSpec((1, DIM), lambda i, e: (e, 0)),
            pl.BlockSpec((1, INTER, DIM), lambda i, e: (e, 0, 0)),
            pl.BlockSpec((1, INTER), lambda i, e: (e, 0)),
        ],
        out_specs=pl.BlockSpec((bm, DIM), lambda i, e: (i, 0)),
        out_shape=jax.ShapeDtypeStruct((t, DIM), jnp.float32),
        scratch_shapes=[pltpu.VMEM((bm, 16), jnp.float32)],
        compiler_params=pltpu.CompilerParams(
            dimension_semantics=("arbitrary", "arbitrary")),
    )(xf, gate_w, gate_b, V1, VB1, V2, VB2, V3, VB3)


def kernel(x, gate_w, gate_b, W1, B1, W2, B2, W3, B3, SW1, SB1, SW2, SB2, SW3, SB3):
    bsz, seq, h = x.shape
    xf = x.reshape(-1, h)
    V1 = jnp.concatenate([W1, SW1.reshape(2, INTER, DIM)], axis=0)
    V3 = jnp.concatenate([W3, SW3.reshape(2, INTER, DIM)], axis=0)
    V2 = jnp.concatenate(
        [W2, SW2.reshape(DIM, 2, INTER).transpose(1, 0, 2)], axis=0)
    VB1 = jnp.concatenate([B1, SB1.reshape(2, INTER)], axis=0)
    VB3 = jnp.concatenate([B3, SB3.reshape(2, INTER)], axis=0)
    VB2 = jnp.concatenate(
        [B2, SB2[None], jnp.zeros((1, DIM), jnp.float32)], axis=0)
    y = _moe_dense(xf, gate_w, gate_b, V1, VB1, V2, VB2, V3, VB3)
    return y.reshape(bsz, seq, h)


# speedup vs baseline: 1.5881x; 1.5881x over previous
import functools

import jax
import jax.numpy as jnp
from jax import lax
from jax.experimental import pallas as pl
from jax.experimental.pallas import tpu as pltpu

E = 8
TOP_K = 2
DIM = 1024
INTER = 512
NV = 10


def _silu(v):
    return v * (1.0 / (1.0 + jnp.exp(-v)))


def _moe_block(x_ref, gw_ref, gb_ref, v1_ref, vb1_ref, v2_ref, vb2_ref,
               v3_ref, vb3_ref, out_ref, comb_ref):
    e = pl.program_id(1)
    xs = x_ref[...]

    @pl.when(e == 0)
    def _gate():
        logits = lax.dot_general(xs, gw_ref[...], (((1,), (1,)), ((), ())),
                                 preferred_element_type=jnp.float32)
        logits = logits + gb_ref[...]
        m = jnp.max(logits, axis=1, keepdims=True)
        ex = jnp.exp(logits - m)
        scores = ex / jnp.sum(ex, axis=1, keepdims=True)
        cols = lax.broadcasted_iota(jnp.int32, scores.shape, 1)
        m1 = jnp.max(scores, axis=1, keepdims=True)
        i1 = jnp.min(jnp.where(scores == m1, cols, E), axis=1, keepdims=True)
        masked = jnp.where(cols == i1, -jnp.inf, scores)
        m2 = jnp.max(masked, axis=1, keepdims=True)
        i2 = jnp.min(jnp.where(masked == m2, cols, E), axis=1, keepdims=True)
        denom = m1 + m2 + 1e-20
        w1 = m1 / denom
        w2 = m2 / denom
        cols16 = lax.broadcasted_iota(jnp.int32, (xs.shape[0], 16), 1)
        comb = (jnp.where(cols16 == i1, w1, 0.0)
                + jnp.where(cols16 == i2, w2, 0.0)
                + jnp.where(cols16 >= E, 1.0, 0.0))
        comb_ref[...] = comb

    h1 = lax.dot_general(xs, v1_ref[0], (((1,), (1,)), ((), ())),
                         preferred_element_type=jnp.float32) + vb1_ref[0]
    h3 = lax.dot_general(xs, v3_ref[0], (((1,), (1,)), ((), ())),
                         preferred_element_type=jnp.float32) + vb3_ref[0]
    h = _silu(h1) * h3
    oe = lax.dot_general(h, v2_ref[0], (((1,), (1,)), ((), ())),
                         preferred_element_type=jnp.float32) + vb2_ref[0]
    cols16 = lax.broadcasted_iota(jnp.int32, comb_ref.shape, 1)
    ce = jnp.sum(jnp.where(cols16 == e, comb_ref[...], 0.0), axis=1,
                 keepdims=True)

    @pl.when(e == 0)
    def _init():
        out_ref[...] = oe * ce

    @pl.when(e != 0)
    def _acc():
        out_ref[...] = out_ref[...] + oe * ce


@functools.partial(jax.jit, static_argnames=("bm",))
def _moe_dense(xf, gate_w, gate_b, V1, VB1, V2, VB2, V3, VB3, bm=1024):
    t = xf.shape[0]
    grid = (t // bm, NV)
    return pl.pallas_call(
        _moe_block,
        grid=grid,
        in_specs=[
            pl.BlockSpec((bm, DIM), lambda i, e: (i, 0)),
            pl.BlockSpec((E, DIM), lambda i, e: (0, 0)),
            pl.BlockSpec((1, E), lambda i, e: (0, 0)),
            pl.BlockSpec((1, INTER, DIM), lambda i, e: (e, 0, 0)),
            pl.BlockSpec((1, 1, INTER), lambda i, e: (e, 0, 0)),
            pl.BlockSpec((1, DIM, INTER), lambda i, e: (e, 0, 0)),
            pl.BlockSpec((1, 1, DIM), lambda i, e: (e, 0, 0)),
            pl.BlockSpec((1, INTER, DIM), lambda i, e: (e, 0, 0)),
            pl.BlockSpec((1, 1, INTER), lambda i, e: (e, 0, 0)),
        ],
        out_specs=pl.BlockSpec((bm, DIM), lambda i, e: (i, 0)),
        out_shape=jax.ShapeDtypeStruct((t, DIM), jnp.float32),
        scratch_shapes=[pltpu.VMEM((bm, 16), jnp.float32)],
        compiler_params=pltpu.CompilerParams(
            dimension_semantics=("arbitrary", "arbitrary")),
    )(xf, gate_w, gate_b.reshape(1, E),
      V1, VB1[:, None], V2, VB2[:, None], V3, VB3[:, None])


def kernel(x, gate_w, gate_b, W1, B1, W2, B2, W3, B3, SW1, SB1, SW2, SB2, SW3, SB3):
    bsz, seq, h = x.shape
    xf = x.reshape(-1, h)
    V1 = jnp.concatenate([W1, SW1.reshape(2, INTER, DIM)], axis=0)
    V3 = jnp.concatenate([W3, SW3.reshape(2, INTER, DIM)], axis=0)
    V2 = jnp.concatenate(
        [W2, SW2.reshape(DIM, 2, INTER).transpose(1, 0, 2)], axis=0)
    VB1 = jnp.concatenate([B1, SB1.reshape(2, INTER)], axis=0)
    VB3 = jnp.concatenate([B3, SB3.reshape(2, INTER)], axis=0)
    VB2 = jnp.concatenate(
        [B2, SB2[None], jnp.zeros((1, DIM), jnp.float32)], axis=0)
    y = _moe_dense(xf, gate_w, gate_b, V1, VB1, V2, VB2, V3, VB3)
    return y.reshape(bsz, seq, h)
